# Initial kernel scaffold; baseline (speedup 1.0000x reference)
import jax
import jax.numpy as jnp
from jax.experimental import pallas as pl


def kernel(queues, x, num, cat, emb_table, W_in, b_in, W_conv, b_conv, W_res, b_res, W_skip, b_skip, W_o1, b_o1, W_o2, b_o2):
    raise NotImplementedError("write your pallas kernel here")


# speedup vs baseline: 1.4857x; 1.4857x over previous
import jax
import jax.numpy as jnp
from jax import lax
from jax.experimental import pallas as pl
from jax.experimental.pallas import tpu as pltpu

_NBLK = 24
_DILC = 8
_BT = 128


def _decoder_kernel(x_ref, num_ref, cat_ref, emb_ref, wint_ref, bin_ref,
                    q_ref, wc0_ref, wc1_ref, bc_ref, wrt_ref, brt_ref,
                    wst_ref, bst_ref, wo1_ref, bo1_ref, wo2_ref, bo2_ref,
                    out_ref, newq_ref, cur_ref, acc_ref):
    i = pl.program_id(1)
    f32 = jnp.float32

    @pl.when(i == 0)
    def _init():
        xv = x_ref[:, :, 0]
        nv = num_ref[:, :, 0]
        idx = cat_ref[:, :, 0]
        oh = (idx == lax.broadcasted_iota(jnp.int32, (1, 1000), 1)).astype(f32)
        emb = jnp.dot(oh, emb_ref[...], preferred_element_type=f32)
        wt = wint_ref[...]
        cur0 = (xv * wt[0:1, :]
                + jnp.dot(nv, wt[1:9, :], preferred_element_type=f32)
                + jnp.dot(emb, wt[9:25, :], preferred_element_type=f32)
                + bin_ref[...])
        cur_ref[...] = cur0
        acc_ref[...] = jnp.zeros_like(acc_ref)

    def _tap(d):
        return lambda: q_ref[0, :, :, 256 - d]

    tap0 = lax.switch(jnp.bitwise_and(i, _DILC - 1),
                      [_tap(1 << k) for k in range(_DILC)])

    cur = cur_ref[...]
    z = (jnp.dot(tap0, wc0_ref[0], preferred_element_type=f32)
         + jnp.dot(cur, wc1_ref[0], preferred_element_type=f32)
         + bc_ref[0])
    fz = jnp.tanh(z[:, :32])
    gz = z[:, 32:]
    gated = fz / (1.0 + jnp.exp(-gz))

    skip = jnp.dot(gated, wst_ref[0], preferred_element_type=f32) + bst_ref[0]
    acc_ref[...] += jnp.dot(jnp.maximum(skip, 0.0), wo1_ref[0],
                            preferred_element_type=f32)
    cur_ref[...] = (jnp.dot(gated, wrt_ref[0], preferred_element_type=f32)
                    + brt_ref[0] + cur)

    newq_ref[0, :, :, 0:256] = q_ref[0]
    newq_ref[0, :, :, 256] = cur

    @pl.when(i == _NBLK - 1)
    def _finalize():
        h = jnp.maximum(acc_ref[...] + bo1_ref[...], 0.0)
        o = jnp.sum(h * wo2_ref[...], axis=1, keepdims=True) + bo2_ref[...]
        out_ref[...] = o[:, :, None]


def kernel(queues, x, num, cat, emb_table, W_in, b_in, W_conv, b_conv,
           W_res, b_res, W_skip, b_skip, W_o1, b_o1, W_o2, b_o2):
    B, C, Tq = queues.shape[1], queues.shape[2], queues.shape[3]
    S = W_skip.shape[1]
    nbt = B // _BT

    wint = W_in.T
    binr = b_in[None, :]
    wc0 = W_conv[:, :, :, 0].transpose(0, 2, 1)
    wc1 = W_conv[:, :, :, 1].transpose(0, 2, 1)
    bc = b_conv[:, None, :]
    wrt = W_res.transpose(0, 2, 1)
    brt = b_res[:, None, :]
    wst = W_skip.transpose(0, 2, 1)
    bst = b_skip[:, None, :]
    wo1 = W_o1.reshape(128, _NBLK, S).transpose(1, 2, 0)
    bo1 = b_o1[None, :]
    bo2 = b_o2[None, :]
    cat32 = cat.astype(jnp.int32)

    grid = (nbt, _NBLK)
    bspec = pl.BlockSpec

    out, newq = pl.pallas_call(
        _decoder_kernel,
        grid=grid,
        in_specs=[
            bspec((_BT, 1, 1), lambda b, i: (b, 0, 0)),
            bspec((_BT, 8, 1), lambda b, i: (b, 0, 0)),
            bspec((_BT, 1, 1), lambda b, i: (b, 0, 0)),
            bspec((1000, 16), lambda b, i: (0, 0)),
            bspec((25, 32), lambda b, i: (0, 0)),
            bspec((1, 32), lambda b, i: (0, 0)),
            bspec((1, _BT, C, Tq), lambda b, i: (i, b, 0, 0)),
            bspec((1, C, 2 * C), lambda b, i: (i, 0, 0)),
            bspec((1, C, 2 * C), lambda b, i: (i, 0, 0)),
            bspec((1, 1, 2 * C), lambda b, i: (i, 0, 0)),
            bspec((1, C, C), lambda b, i: (i, 0, 0)),
            bspec((1, 1, C), lambda b, i: (i, 0, 0)),
            bspec((1, C, C), lambda b, i: (i, 0, 0)),
            bspec((1, 1, C), lambda b, i: (i, 0, 0)),
            bspec((1, S, 128), lambda b, i: (i, 0, 0)),
            bspec((1, 128), lambda b, i: (0, 0)),
            bspec((1, 128), lambda b, i: (0, 0)),
            bspec((1, 1), lambda b, i: (0, 0)),
        ],
        out_specs=[
            bspec((_BT, 1, 1), lambda b, i: (b, 0, 0)),
            bspec((1, _BT, C, Tq + 1), lambda b, i: (i, b, 0, 0)),
        ],
        out_shape=[
            jax.ShapeDtypeStruct((B, 1, 1), jnp.float32),
            jax.ShapeDtypeStruct((_NBLK, B, C, Tq + 1), jnp.float32),
        ],
        scratch_shapes=[
            pltpu.VMEM((_BT, C), jnp.float32),
            pltpu.VMEM((_BT, 128), jnp.float32),
        ],
        compiler_params=pltpu.CompilerParams(
            dimension_semantics=("parallel", "arbitrary"),
        ),
        name="wave_decoder_step",
    )(x, num, cat32, emb_table, wint, binr, queues, wc0, wc1, bc,
      wrt, brt, wst, bst, wo1, bo1, W_o2, bo2)

    return out, newq


# speedup vs baseline: 1.7246x; 1.1608x over previous
import jax
import jax.numpy as jnp
from jax import lax
from jax.experimental import pallas as pl
from jax.experimental.pallas import tpu as pltpu

_NBLK = 24
_DILC = 8


def _decoder_kernel(x_ref, num_ref, cat_ref, embt_ref, win_ref, bin_ref,
                    q_ref, wc0_ref, wc1_ref, bc_ref, wrt_ref, brt_ref,
                    wst_ref, bst_ref, wo1_ref, bo1_ref, wo2_ref, bo2_ref,
                    out_ref, newq_hbm, tr_ref, cur_ref, acc_ref, sems):
    i = pl.program_id(0)
    f32 = jnp.float32
    B = x_ref.shape[0]
    C = cur_ref.shape[0]
    Tq = q_ref.shape[2] // C

    @pl.when(i == 0)
    def _init():
        xT = jnp.transpose(x_ref[:, :, 0], (1, 0))
        numT = jnp.transpose(num_ref[:, :, 0], (1, 0))
        catT = jnp.transpose(cat_ref[:, :, 0], (1, 0))
        ohT = (lax.broadcasted_iota(jnp.int32, (1000, 1), 0) == catT).astype(f32)
        embT = jnp.dot(embt_ref[...], ohT, preferred_element_type=f32)
        w = win_ref[...]
        cur0 = (w[:, 0:1] * xT
                + jnp.dot(w[:, 1:9], numT, preferred_element_type=f32)
                + jnp.dot(w[:, 9:25], embT, preferred_element_type=f32)
                + bin_ref[...])
        cur_ref[...] = cur0
        acc_ref[...] = jnp.zeros_like(acc_ref)

    qT = jnp.transpose(q_ref[0], (1, 0))
    tr_ref[:, 0:Tq, :] = qT.reshape(C, Tq, B)
    cur = cur_ref[...]
    tr_ref[:, Tq, :] = cur

    def _tap(d):
        return lambda: tr_ref[:, Tq - d, :]

    tapT = lax.switch(jnp.bitwise_and(i, _DILC - 1),
                      [_tap(1 << k) for k in range(_DILC)])

    zT = (jnp.dot(wc0_ref[0], tapT, preferred_element_type=f32)
          + jnp.dot(wc1_ref[0], cur, preferred_element_type=f32)
          + bc_ref[0])
    fz = jnp.tanh(zT[:C, :])
    gz = zT[C:, :]
    gatedT = fz / (1.0 + jnp.exp(-gz))

    skipT = jnp.dot(wst_ref[0], gatedT, preferred_element_type=f32) + bst_ref[0]
    acc_ref[...] += jnp.dot(wo1_ref[0], jnp.maximum(skipT, 0.0),
                            preferred_element_type=f32)
    cur_ref[...] = (jnp.dot(wrt_ref[0], gatedT, preferred_element_type=f32)
                    + brt_ref[0] + cur)

    for c in range(C):
        pltpu.make_async_copy(tr_ref.at[c], newq_hbm.at[i, :, c, :],
                              sems.at[c]).start()
    for c in range(C):
        pltpu.make_async_copy(tr_ref.at[c], newq_hbm.at[i, :, c, :],
                              sems.at[c]).wait()

    @pl.when(i == _NBLK - 1)
    def _finalize():
        hT = jnp.maximum(acc_ref[...] + bo1_ref[...], 0.0)
        outT = jnp.dot(wo2_ref[...], hT, preferred_element_type=f32) + bo2_ref[...]
        out_ref[...] = outT[None]


def kernel(queues, x, num, cat, emb_table, W_in, b_in, W_conv, b_conv,
           W_res, b_res, W_skip, b_skip, W_o1, b_o1, W_o2, b_o2):
    B, C, Tq = queues.shape[1], queues.shape[2], queues.shape[3]
    S = W_skip.shape[1]

    q2 = queues.reshape(_NBLK, B, C * Tq)
    wc0 = W_conv[:, :, :, 0]
    wc1 = W_conv[:, :, :, 1]
    bc = b_conv[:, :, None]
    brt = b_res[:, :, None]
    bst = b_skip[:, :, None]
    wo1 = W_o1.reshape(128, _NBLK, S).transpose(1, 0, 2)
    binr = b_in[:, None]
    bo1 = b_o1[:, None]
    bo2 = b_o2[:, None]
    embt = emb_table.T
    cat32 = cat.astype(jnp.int32)

    bspec = pl.BlockSpec

    out, newq = pl.pallas_call(
        _decoder_kernel,
        grid=(_NBLK,),
        in_specs=[
            bspec((B, 1, 1), lambda i: (0, 0, 0)),
            bspec((B, 8, 1), lambda i: (0, 0, 0)),
            bspec((B, 1, 1), lambda i: (0, 0, 0)),
            bspec((16, 1000), lambda i: (0, 0)),
            bspec((C, 25), lambda i: (0, 0)),
            bspec((C, 1), lambda i: (0, 0)),
            bspec((1, B, C * Tq), lambda i: (i, 0, 0)),
            bspec((1, 2 * C, C), lambda i: (i, 0, 0)),
            bspec((1, 2 * C, C), lambda i: (i, 0, 0)),
            bspec((1, 2 * C, 1), lambda i: (i, 0, 0)),
            bspec((1, C, C), lambda i: (i, 0, 0)),
            bspec((1, C, 1), lambda i: (i, 0, 0)),
            bspec((1, S, C), lambda i: (i, 0, 0)),
            bspec((1, S, 1), lambda i: (i, 0, 0)),
            bspec((1, 128, S), lambda i: (i, 0, 0)),
            bspec((128, 1), lambda i: (0, 0)),
            bspec((1, 128), lambda i: (0, 0)),
            bspec((1, 1), lambda i: (0, 0)),
        ],
        out_specs=[
            bspec((1, 1, B), lambda i: (0, 0, 0)),
            bspec(memory_space=pl.ANY),
        ],
        out_shape=[
            jax.ShapeDtypeStruct((1, 1, B), jnp.float32),
            jax.ShapeDtypeStruct((_NBLK, Tq + 1, C, B), jnp.float32),
        ],
        scratch_shapes=[
            pltpu.VMEM((C, Tq + 1, B), jnp.float32),
            pltpu.VMEM((C, B), jnp.float32),
            pltpu.VMEM((128, B), jnp.float32),
            pltpu.SemaphoreType.DMA((C,)),
        ],
        compiler_params=pltpu.CompilerParams(
            dimension_semantics=("arbitrary",),
            vmem_limit_bytes=48 * 1024 * 1024,
        ),
        name="wave_decoder_step",
    )(x, num, cat32, embt, W_in, binr, q2, wc0, wc1, bc,
      W_res, brt, W_skip, bst, wo1, bo1, W_o2, bo2)

    return out.reshape(B, 1, 1), jnp.transpose(newq, (0, 3, 2, 1))
